# baseline (speedup 1.0000x reference)
import functools

import jax
import jax.numpy as jnp
from jax import lax
from jax.experimental import pallas as pl
from jax.experimental.pallas import tpu as pltpu
from jax.experimental.pallas import tpu_sc as plsc

H = 128
H3 = 3 * H
EPS = 1e-08

NW = 32
NB = 160
NBKT = 63
ROUNDS = 2
NOUT = NBKT * NB
CHW = 2000


def _clip(x):
    return jnp.clip(x, -100.0, 100.0)


def _mlp1(s2d, W1, b1, W2, b2):
    n = s2d.shape[0]
    bs = 400

    def body(s_ref, w1_ref, b1_ref, w2_ref, b2_ref, o_ref):
        h = jnp.dot(s_ref[...], w1_ref[...], preferred_element_type=jnp.float32)
        h = h + b1_ref[...]
        h = h * jax.nn.sigmoid(h)
        o = jnp.dot(h, w2_ref[...], preferred_element_type=jnp.float32)
        o_ref[...] = o + b2_ref[...]

    return pl.pallas_call(
        body,
        grid=(n // bs,),
        in_specs=[
            pl.BlockSpec((bs, H), lambda i: (i, 0)),
            pl.BlockSpec((H, H), lambda i: (0, 0)),
            pl.BlockSpec((1, H), lambda i: (0, 0)),
            pl.BlockSpec((H, H3), lambda i: (0, 0)),
            pl.BlockSpec((1, H3), lambda i: (0, 0)),
        ],
        out_specs=pl.BlockSpec((bs, H3), lambda i: (i, 0)),
        out_shape=jax.ShapeDtypeStruct((n, H3), jnp.float32),
    )(s2d, W1, b1.reshape(1, H), W2, b2.reshape(1, H3))


def _sc_edges(senders, receivers, dx, dy, dz, w2d, x2d, v2d):
    e = senders.shape[0]

    @functools.partial(
        pl.kernel,
        out_type=jax.ShapeDtypeStruct((NOUT, 4 * H), jnp.float32),
        mesh=plsc.VectorSubcoreMesh(core_axis_name="c", subcore_axis_name="s"),
        scratch_types=[
            pltpu.VMEM((NB, 4 * H), jnp.float32),
            pltpu.VMEM((CHW,), jnp.int32),
            pltpu.VMEM((CHW,), jnp.int32),
            pltpu.VMEM((CHW,), jnp.float32),
            pltpu.VMEM((CHW,), jnp.float32),
            pltpu.VMEM((CHW,), jnp.float32),
            pltpu.VMEM((16, H3), jnp.float32),
            pltpu.VMEM((16, H3), jnp.float32),
            pltpu.VMEM((16, H3), jnp.float32),
            pltpu.SMEM((48,), jnp.int32),
            pltpu.SMEM((48,), jnp.int32),
            pltpu.SMEM((48,), jnp.int32),
            pltpu.SMEM((48,), jnp.float32),
            pltpu.SMEM((48,), jnp.float32),
            pltpu.SMEM((48,), jnp.float32),
            pltpu.SMEM((8,), jnp.int32),
        ],
    )
    def k(snd_hbm, rcv_hbm, dx_hbm, dy_hbm, dz_hbm, w_hbm, x_hbm, v_hbm,
          out_hbm, acc, sbuf, rbuf, dbx, dby, dbz, wg, xg, vg,
          qs, qr, qe, qdx, qdy, qdz, qn):
        wid = lax.axis_index("s") * 2 + lax.axis_index("c")
        iot = lax.iota(jnp.int32, 16)
        zero16 = jnp.zeros((16,), jnp.float32)
        zero_i = jnp.zeros((16,), jnp.int32)
        perms = [iot ^ sh for sh in (8, 4, 2, 1)]

        def any_hit(msk):
            ones = jnp.where(msk, 1, 0).astype(jnp.int32)
            for p in perms:
                ones = ones | ones.at[p].get(mode="promise_in_bounds")
            return ones[0]

        def gather_idx(q):
            v = zero_i
            for j in range(16):
                v = jnp.where(iot == j, jnp.full((16,), q[j], jnp.int32), v)
            return v

        def flush(cnt):
            idxe = gather_idx(qe)
            idxr = gather_idx(qr)
            pltpu.sync_copy(w_hbm.at[idxe], wg)
            pltpu.sync_copy(x_hbm.at[idxr], xg)
            pltpu.sync_copy(v_hbm.at[idxr], vg)

            def compute(j, c0):
                row = qs[j]
                dvecs = [jnp.full((16,), q[j], jnp.float32)
                         for q in (qdx, qdy, qdz)]
                for c in range(8):
                    co = c * 16
                    wv0 = wg[j, pl.ds(co, 16)]
                    wv1 = wg[j, pl.ds(H + co, 16)]
                    wv2 = wg[j, pl.ds(2 * H + co, 16)]
                    xv0 = xg[j, pl.ds(co, 16)]
                    xv1 = xg[j, pl.ds(H + co, 16)]
                    xv2 = xg[j, pl.ds(2 * H + co, 16)]
                    dv1 = wv1 * xv1
                    dv2 = wv2 * xv2
                    plsc.addupdate(acc.at[row, pl.ds(co, 16)], wv0 * xv0)
                    for kk in range(3):
                        vj = vg[j, pl.ds(kk * H + co, 16)]
                        plsc.addupdate(
                            acc.at[row, pl.ds(H + kk * H + co, 16)],
                            dv1 * dvecs[kk] + dv2 * vj)
                return c0

            lax.fori_loop(0, cnt, compute, 0)
            for j in range(16):
                qs[j] = qs[16 + j]
                qr[j] = qr[16 + j]
                qe[j] = qe[16 + j]
                qdx[j] = qdx[16 + j]
                qdy[j] = qdy[16 + j]
                qdz[j] = qdz[16 + j]

        for r in range(ROUNDS):
            b = r * NW + wid

            @pl.when(b < NBKT)
            def _():
                lo = b * NB
                hiv = jnp.full((16,), lo + NB, jnp.int32)
                lov = jnp.full((16,), lo, jnp.int32)

                def zrow(i, c):
                    for cc in range(4 * H // 16):
                        acc[i, pl.ds(cc * 16, 16)] = zero16
                    return c

                lax.fori_loop(0, NB, zrow, 0)
                for j in range(48):
                    qs[j] = jnp.int32(0)
                    qr[j] = jnp.int32(0)
                    qe[j] = jnp.int32(0)
                    qdx[j] = jnp.float32(0)
                    qdy[j] = jnp.float32(0)
                    qdz[j] = jnp.float32(0)
                qn[0] = jnp.int32(0)

                def chunk_body(ch, hn):
                    pltpu.sync_copy(snd_hbm.at[pl.ds(ch * CHW, CHW)], sbuf)
                    pltpu.sync_copy(rcv_hbm.at[pl.ds(ch * CHW, CHW)], rbuf)
                    pltpu.sync_copy(dx_hbm.at[pl.ds(ch * CHW, CHW)], dbx)
                    pltpu.sync_copy(dy_hbm.at[pl.ds(ch * CHW, CHW)], dby)
                    pltpu.sync_copy(dz_hbm.at[pl.ds(ch * CHW, CHW)], dbz)

                    def grp_body(i, c):
                        sv = sbuf[pl.ds(i * 16, 16)]
                        msk = (sv >= lov) & (sv < hiv)

                        @pl.when(any_hit(msk) > 0)
                        def _():
                            rv = rbuf[pl.ds(i * 16, 16)]
                            dvx = dbx[pl.ds(i * 16, 16)]
                            dvy = dby[pl.ds(i * 16, 16)]
                            dvz = dbz[pl.ds(i * 16, 16)]
                            base_e = ch * CHW + i * 16
                            for j in range(16):
                                s = sv[j]

                                @pl.when((s >= lo) & (s < lo + NB))
                                def _(j=j, s=s):
                                    cur = qn[0]
                                    qs[cur] = s - lo
                                    qr[cur] = rv[j]
                                    qe[cur] = base_e + j
                                    qdx[cur] = dvx[j]
                                    qdy[cur] = dvy[j]
                                    qdz[cur] = dvz[j]
                                    qn[0] = cur + 1

                            cnt = qn[0]

                            @pl.when(cnt >= 16)
                            def _():
                                flush(16)
                                qn[0] = cnt - 16

                        return c

                    return lax.fori_loop(0, CHW // 16, grp_body, hn)

                lax.fori_loop(0, e // CHW, chunk_body, jnp.int32(0))
                tailn = qn[0]
                pl.when(tailn > 0)(lambda: flush(tailn))
                pltpu.sync_copy(acc, out_hbm.at[pl.ds(lo, NB)])

    return k(senders, receivers, dx, dy, dz, w2d, x2d, v2d)


def _update(s2d, v2d, dsum, dvsum, W_vmix, W_mix1, b_mix1, W_mix2, b_mix2):
    n = s2d.shape[0]
    bs = 400

    def body(s_ref, v_ref, ds_ref, dv_ref, wv_ref, w1_ref, b1_ref, w2_ref,
             b2_ref, so_ref, vo_ref):
        s1 = s_ref[...] + _clip(ds_ref[...])
        v1 = v_ref[...] + _clip(dv_ref[...])
        wv = wv_ref[...]
        v1k = [v1[:, kk * H:(kk + 1) * H] for kk in range(3)]
        vm = [jnp.dot(vk, wv, preferred_element_type=jnp.float32) for vk in v1k]
        v_l = [m[:, :H] for m in vm]
        v_r = [m[:, H:] for m in vm]
        nsq = v_r[0] * v_r[0] + v_r[1] * v_r[1] + v_r[2] * v_r[2]
        v_norm = jnp.sqrt(nsq + EPS)
        w1 = w1_ref[...]
        h = (jnp.dot(s1, w1[:H, :], preferred_element_type=jnp.float32)
             + jnp.dot(v_norm, w1[H:, :], preferred_element_type=jnp.float32)
             + b1_ref[...])
        h = h * jax.nn.sigmoid(h)
        m = jnp.dot(h, w2_ref[...], preferred_element_type=jnp.float32)
        m = m + b2_ref[...]
        ds2 = m[:, :H]
        dvu_g = m[:, H:2 * H]
        dsv_g = m[:, 2 * H:]
        dot_rl = v_r[0] * v_l[0] + v_r[1] * v_l[1] + v_r[2] * v_l[2]
        so_ref[...] = s1 + _clip(ds2 + dsv_g * dot_rl)
        vo_ref[...] = jnp.concatenate(
            [v1k[kk] + _clip(v_l[kk] * dvu_g) for kk in range(3)], axis=1)

    return pl.pallas_call(
        body,
        grid=(n // bs,),
        in_specs=[
            pl.BlockSpec((bs, H), lambda i: (i, 0)),
            pl.BlockSpec((bs, H3), lambda i: (i, 0)),
            pl.BlockSpec((bs, H), lambda i: (i, 0)),
            pl.BlockSpec((bs, H3), lambda i: (i, 0)),
            pl.BlockSpec((H, 2 * H), lambda i: (0, 0)),
            pl.BlockSpec((2 * H, H), lambda i: (0, 0)),
            pl.BlockSpec((1, H), lambda i: (0, 0)),
            pl.BlockSpec((H, H3), lambda i: (0, 0)),
            pl.BlockSpec((1, H3), lambda i: (0, 0)),
        ],
        out_specs=[
            pl.BlockSpec((bs, H), lambda i: (i, 0)),
            pl.BlockSpec((bs, H3), lambda i: (i, 0)),
        ],
        out_shape=[
            jax.ShapeDtypeStruct((n, H), jnp.float32),
            jax.ShapeDtypeStruct((n, H3), jnp.float32),
        ],
    )(s2d, v2d, dsum, dvsum, W_vmix, W_mix1, b_mix1.reshape(1, H), W_mix2,
      b_mix2.reshape(1, H3))


def kernel(s, v, dir_ij, Wij, senders, receivers, W_int1, b_int1, W_int2,
           b_int2, W_vmix, W_mix1, b_mix1, W_mix2, b_mix2):
    n = s.shape[0]
    e = senders.shape[0]
    s2d = s.reshape(n, H)
    v2d = v.reshape(n, H3)
    w2d = Wij.reshape(e, H3)
    x2d = _mlp1(s2d, W_int1, b_int1, W_int2, b_int2)

    agg = _sc_edges(senders, receivers, dir_ij[:, 0], dir_ij[:, 1],
                    dir_ij[:, 2], w2d, x2d, v2d)
    dsum = agg[:n, :H]
    dvsum = agg[:n, H:]

    s_out, v_out = _update(s2d, v2d, dsum, dvsum, W_vmix, W_mix1, b_mix1,
                           W_mix2, b_mix2)
    return (s_out.reshape(n, 1, H), v_out.reshape(n, 3, H))


# speedup vs baseline: 2.1219x; 2.1219x over previous
import functools

import jax
import jax.numpy as jnp
from jax import lax
from jax.experimental import pallas as pl
from jax.experimental.pallas import tpu as pltpu
from jax.experimental.pallas import tpu_sc as plsc

H = 128
H3 = 3 * H
EPS = 1e-08

NW = 32
NB = 160
NBKT = 63
ROUNDS = 2
NOUT = NBKT * NB
CHW = 2000


def _clip(x):
    return jnp.clip(x, -100.0, 100.0)


def _bucket(snd):
    return ((snd >> 5) * 13108) >> 16


def _mlp1(s2d, W1, b1, W2, b2):
    n = s2d.shape[0]
    bs = 400

    def body(s_ref, w1_ref, b1_ref, w2_ref, b2_ref, o_ref):
        h = jnp.dot(s_ref[...], w1_ref[...], preferred_element_type=jnp.float32)
        h = h + b1_ref[...]
        h = h * jax.nn.sigmoid(h)
        o = jnp.dot(h, w2_ref[...], preferred_element_type=jnp.float32)
        o_ref[...] = o + b2_ref[...]

    return pl.pallas_call(
        body,
        grid=(n // bs,),
        in_specs=[
            pl.BlockSpec((bs, H), lambda i: (i, 0)),
            pl.BlockSpec((H, H), lambda i: (0, 0)),
            pl.BlockSpec((1, H), lambda i: (0, 0)),
            pl.BlockSpec((H, H3), lambda i: (0, 0)),
            pl.BlockSpec((1, H3), lambda i: (0, 0)),
        ],
        out_specs=pl.BlockSpec((bs, H3), lambda i: (i, 0)),
        out_shape=jax.ShapeDtypeStruct((n, H3), jnp.float32),
    )(s2d, W1, b1.reshape(1, H), W2, b2.reshape(1, H3))


def _sc_mesh():
    return plsc.VectorSubcoreMesh(core_axis_name="c", subcore_axis_name="s")


def _wid():
    return lax.axis_index("s") * 2 + lax.axis_index("c")


def _sc_hist(senders):
    e = senders.shape[0]
    epw = e // NW

    @functools.partial(
        pl.kernel,
        out_type=jax.ShapeDtypeStruct((NW * 64,), jnp.int32),
        mesh=_sc_mesh(),
        scratch_types=[
            pltpu.VMEM((64,), jnp.int32),
            pltpu.VMEM((CHW,), jnp.int32),
        ],
    )
    def k(snd_hbm, out_hbm, hist, pbuf):
        wid = _wid()
        iot = lax.iota(jnp.int32, 16)
        one_i = jnp.ones((16,), jnp.int32)
        zero_i = jnp.zeros((16,), jnp.int32)
        for g in range(4):
            hist[pl.ds(g * 16, 16)] = zero_i
        for ch in range(epw // CHW):
            pltpu.sync_copy(
                snd_hbm.at[pl.ds(wid * epw + ch * CHW, CHW)], pbuf)

            def vec_body(i, c):
                sv = pbuf[pl.ds(i * 16, 16)]
                for j in range(16):
                    bb = _bucket(sv[j])
                    oh = jnp.where(iot == (bb & 15), one_i, zero_i)
                    plsc.addupdate(hist.at[pl.ds((bb >> 4) * 16, 16)], oh)
                return c

            lax.fori_loop(0, CHW // 16, vec_body, 0)
        pltpu.sync_copy(hist, out_hbm.at[pl.ds(wid * 64, 64)])

    return k(senders)


def _sc_scatter(senders, meta2d, counts):
    e = senders.shape[0]
    epw = e // NW

    @functools.partial(
        pl.kernel,
        out_type=jax.ShapeDtypeStruct((e + NBKT * 8 + 16, 16), jnp.float32),
        mesh=_sc_mesh(),
        scratch_types=[
            pltpu.VMEM((NW * 64,), jnp.int32),
            pltpu.VMEM((CHW,), jnp.int32),
            pltpu.SMEM((64,), jnp.int32),
            pltpu.SemaphoreType.DMA,
        ],
    )
    def k(snd_hbm, meta_hbm, cnt_hbm, out_hbm, cbuf, pbuf, smem, sem):
        wid = _wid()
        zero_i = jnp.zeros((16,), jnp.int32)
        pltpu.sync_copy(cnt_hbm, cbuf)
        T = [zero_i] * 4
        PS = [zero_i] * 4
        for t in range(NW):
            before = t < wid
            for g in range(4):
                r = cbuf[pl.ds(t * 64 + g * 16, 16)]
                T[g] = T[g] + r
                PS[g] = PS[g] + jnp.where(before, r, zero_i)
        s_run = jnp.int32(0)
        for b in range(NBKT):
            g, l = b >> 4, b & 15
            smem[b] = s_run + PS[g][l]
            s_run = s_run + (((T[g][l] + 7) >> 3) << 3)

        def drain():
            pltpu.make_async_copy(
                meta_hbm.at[0], out_hbm.at[0], sem).wait()

        for ch in range(epw // CHW):
            base_c = wid * epw + ch * CHW
            pltpu.sync_copy(snd_hbm.at[pl.ds(base_c, CHW)], pbuf)

            def vec_body(i, c):
                first = (ch == 0) & (i == 0)

                @pl.when(jnp.logical_not(first))
                def _():
                    for _ in range(16):
                        drain()

                sv = pbuf[pl.ds(i * 16, 16)]
                for j in range(16):
                    bb = _bucket(sv[j])
                    cur = smem[bb]
                    smem[bb] = cur + 1
                    src = base_c + i * 16 + j
                    pltpu.async_copy(
                        meta_hbm.at[src], out_hbm.at[cur], sem)
                return c

            lax.fori_loop(0, CHW // 16, vec_body, 0)
        for _ in range(16):
            drain()

    return k(senders, meta2d, counts)


def _sc_main(sorted2d, counts, x2d, v2d, w2d):
    e = w2d.shape[0]
    n2 = x2d.shape[0]

    @functools.partial(
        pl.kernel,
        out_type=jax.ShapeDtypeStruct((NOUT, 4 * H), jnp.float32),
        mesh=_sc_mesh(),
        scratch_types=[
            pltpu.VMEM((NB, 4 * H), jnp.float32),
            pltpu.VMEM((NW * 64,), jnp.int32),
            pltpu.VMEM((16, 16), jnp.float32),
            pltpu.VMEM((16, H3), jnp.float32),
            pltpu.VMEM((16, H3), jnp.float32),
            pltpu.VMEM((16, H3), jnp.float32),
            pltpu.SMEM((128,), jnp.int32),
        ],
    )
    def k(rec_hbm, cnt_hbm, x_hbm, v_hbm, w_hbm, out_hbm,
          acc, cbuf, recbuf, wg, xg, vg, smem):
        wid = _wid()
        iot = lax.iota(jnp.int32, 16)
        zero16 = jnp.zeros((16,), jnp.float32)
        zero_i = jnp.zeros((16,), jnp.int32)

        pltpu.sync_copy(cnt_hbm, cbuf)
        T = [zero_i] * 4
        for t in range(NW):
            for g in range(4):
                T[g] = T[g] + cbuf[pl.ds(t * 64 + g * 16, 16)]
        s_run = jnp.int32(0)
        for b in range(NBKT):
            smem[b] = s_run >> 3
            smem[64 + b] = T[b >> 4][b & 15]
            s_run = s_run + (((T[b >> 4][b & 15] + 7) >> 3) << 3)

        for r in range(ROUNDS):
            b = r * NW + wid

            @pl.when(b < NBKT)
            def _():
                base = b * NB

                def zrow(i, c):
                    for cc in range(4 * H // 16):
                        acc[i, pl.ds(cc * 16, 16)] = zero16
                    return c

                lax.fori_loop(0, NB, zrow, 0)
                lo8 = smem[b]
                cnt = smem[64 + b]
                nch = (cnt + 15) >> 4

                def chunk_body(ch, c):
                    cbase = (lo8 + ch * 2) * 8
                    pltpu.sync_copy(
                        rec_hbm.at[pl.ds(cbase, 16)], recbuf)
                    idxe = zero_i
                    idxr = zero_i
                    for j in range(16):
                        recj = recbuf[j, pl.ds(0, 16)]
                        ei = jnp.full((16,), jnp.int32(recj[5]), jnp.int32)
                        ri = jnp.full((16,), jnp.int32(recj[1]), jnp.int32)
                        idxe = jnp.where(iot == j, ei, idxe)
                        idxr = jnp.where(iot == j, ri, idxr)
                    idxe = jnp.clip(idxe, 0, e - 1)
                    idxr = jnp.clip(idxr, 0, n2 - 1)
                    pltpu.sync_copy(w_hbm.at[idxe], wg)
                    pltpu.sync_copy(x_hbm.at[idxr], xg)
                    pltpu.sync_copy(v_hbm.at[idxr], vg)
                    nvalid = jnp.minimum(cnt - ch * 16, 16)

                    def compute(j, c0):
                        recj = recbuf[j, pl.ds(0, 16)]
                        row = jnp.int32(recj[0]) - base
                        dvecs = [jnp.full((16,), recj[2 + kk], jnp.float32)
                                 for kk in range(3)]
                        for c in range(8):
                            co = c * 16
                            wv0 = wg[j, pl.ds(co, 16)]
                            wv1 = wg[j, pl.ds(H + co, 16)]
                            wv2 = wg[j, pl.ds(2 * H + co, 16)]
                            xv0 = xg[j, pl.ds(co, 16)]
                            xv1 = xg[j, pl.ds(H + co, 16)]
                            xv2 = xg[j, pl.ds(2 * H + co, 16)]
                            dv1 = wv1 * xv1
                            dv2 = wv2 * xv2
                            plsc.addupdate(
                                acc.at[row, pl.ds(co, 16)], wv0 * xv0)
                            for kk in range(3):
                                vj = vg[j, pl.ds(kk * H + co, 16)]
                                plsc.addupdate(
                                    acc.at[row, pl.ds(H + kk * H + co, 16)],
                                    dv1 * dvecs[kk] + dv2 * vj)
                        return c0

                    lax.fori_loop(0, nvalid, compute, 0)
                    return c

                lax.fori_loop(0, nch, chunk_body, 0)
                pltpu.sync_copy(acc, out_hbm.at[pl.ds(base, NB)])

    return k(sorted2d, counts, x2d, v2d, w2d)


def _update(s2d, v2d, dsum, dvsum, W_vmix, W_mix1, b_mix1, W_mix2, b_mix2):
    n = s2d.shape[0]
    bs = 400

    def body(s_ref, v_ref, ds_ref, dv_ref, wv_ref, w1_ref, b1_ref, w2_ref,
             b2_ref, so_ref, vo_ref):
        s1 = s_ref[...] + _clip(ds_ref[...])
        v1 = v_ref[...] + _clip(dv_ref[...])
        wv = wv_ref[...]
        v1k = [v1[:, kk * H:(kk + 1) * H] for kk in range(3)]
        vm = [jnp.dot(vk, wv, preferred_element_type=jnp.float32) for vk in v1k]
        v_l = [m[:, :H] for m in vm]
        v_r = [m[:, H:] for m in vm]
        nsq = v_r[0] * v_r[0] + v_r[1] * v_r[1] + v_r[2] * v_r[2]
        v_norm = jnp.sqrt(nsq + EPS)
        w1 = w1_ref[...]
        h = (jnp.dot(s1, w1[:H, :], preferred_element_type=jnp.float32)
             + jnp.dot(v_norm, w1[H:, :], preferred_element_type=jnp.float32)
             + b1_ref[...])
        h = h * jax.nn.sigmoid(h)
        m = jnp.dot(h, w2_ref[...], preferred_element_type=jnp.float32)
        m = m + b2_ref[...]
        ds2 = m[:, :H]
        dvu_g = m[:, H:2 * H]
        dsv_g = m[:, 2 * H:]
        dot_rl = v_r[0] * v_l[0] + v_r[1] * v_l[1] + v_r[2] * v_l[2]
        so_ref[...] = s1 + _clip(ds2 + dsv_g * dot_rl)
        vo_ref[...] = jnp.concatenate(
            [v1k[kk] + _clip(v_l[kk] * dvu_g) for kk in range(3)], axis=1)

    return pl.pallas_call(
        body,
        grid=(n // bs,),
        in_specs=[
            pl.BlockSpec((bs, H), lambda i: (i, 0)),
            pl.BlockSpec((bs, H3), lambda i: (i, 0)),
            pl.BlockSpec((bs, H), lambda i: (i, 0)),
            pl.BlockSpec((bs, H3), lambda i: (i, 0)),
            pl.BlockSpec((H, 2 * H), lambda i: (0, 0)),
            pl.BlockSpec((2 * H, H), lambda i: (0, 0)),
            pl.BlockSpec((1, H), lambda i: (0, 0)),
            pl.BlockSpec((H, H3), lambda i: (0, 0)),
            pl.BlockSpec((1, H3), lambda i: (0, 0)),
        ],
        out_specs=[
            pl.BlockSpec((bs, H), lambda i: (i, 0)),
            pl.BlockSpec((bs, H3), lambda i: (i, 0)),
        ],
        out_shape=[
            jax.ShapeDtypeStruct((n, H), jnp.float32),
            jax.ShapeDtypeStruct((n, H3), jnp.float32),
        ],
    )(s2d, v2d, dsum, dvsum, W_vmix, W_mix1, b_mix1.reshape(1, H), W_mix2,
      b_mix2.reshape(1, H3))


def kernel(s, v, dir_ij, Wij, senders, receivers, W_int1, b_int1, W_int2,
           b_int2, W_vmix, W_mix1, b_mix1, W_mix2, b_mix2):
    n = s.shape[0]
    e = senders.shape[0]
    s2d = s.reshape(n, H)
    v2d = v.reshape(n, H3)
    w2d = Wij.reshape(e, H3)

    x2d = _mlp1(s2d, W_int1, b_int1, W_int2, b_int2)

    meta = jnp.concatenate(
        [senders.astype(jnp.float32)[:, None],
         receivers.astype(jnp.float32)[:, None], dir_ij,
         jnp.arange(e, dtype=jnp.float32)[:, None],
         jnp.zeros((e, 10), jnp.float32)], axis=1)
    counts = _sc_hist(senders)
    sorted2d = _sc_scatter(senders, meta, counts)
    agg = _sc_main(sorted2d, counts, x2d, v2d, w2d)
    dsum = agg[:n, :H]
    dvsum = agg[:n, H:]

    s_out, v_out = _update(s2d, v2d, dsum, dvsum, W_vmix, W_mix1, b_mix1,
                           W_mix2, b_mix2)
    return (s_out.reshape(n, 1, H), v_out.reshape(n, 3, H))


# speedup vs baseline: 4.1388x; 1.9505x over previous
import functools

import jax
import jax.numpy as jnp
from jax import lax
from jax.experimental import pallas as pl
from jax.experimental.pallas import tpu as pltpu
from jax.experimental.pallas import tpu_sc as plsc

H = 128
H3 = 3 * H
EPS = 1e-08

NW = 32
NB = 160
NBKT = 63
ROUNDS = 2
NOUT = NBKT * NB
CHW = 2000


def _clip(x):
    return jnp.clip(x, -100.0, 100.0)


def _bucket(snd):
    return ((snd >> 5) * 13108) >> 16


def _mlp1(s2d, W1, b1, W2, b2):
    n = s2d.shape[0]
    bs = 400

    def body(s_ref, w1_ref, b1_ref, w2_ref, b2_ref, o_ref):
        h = jnp.dot(s_ref[...], w1_ref[...], preferred_element_type=jnp.float32)
        h = h + b1_ref[...]
        h = h * jax.nn.sigmoid(h)
        o = jnp.dot(h, w2_ref[...], preferred_element_type=jnp.float32)
        o_ref[...] = o + b2_ref[...]

    return pl.pallas_call(
        body,
        grid=(n // bs,),
        in_specs=[
            pl.BlockSpec((bs, H), lambda i: (i, 0)),
            pl.BlockSpec((H, H), lambda i: (0, 0)),
            pl.BlockSpec((1, H), lambda i: (0, 0)),
            pl.BlockSpec((H, H3), lambda i: (0, 0)),
            pl.BlockSpec((1, H3), lambda i: (0, 0)),
        ],
        out_specs=pl.BlockSpec((bs, H3), lambda i: (i, 0)),
        out_shape=jax.ShapeDtypeStruct((n, H3), jnp.float32),
    )(s2d, W1, b1.reshape(1, H), W2, b2.reshape(1, H3))


def _sc_mesh():
    return plsc.VectorSubcoreMesh(core_axis_name="c", subcore_axis_name="s")


def _wid():
    return lax.axis_index("s") * 2 + lax.axis_index("c")


def _sc_hist(senders):
    e = senders.shape[0]
    epw = e // NW

    @functools.partial(
        pl.kernel,
        out_type=jax.ShapeDtypeStruct((NW * 64,), jnp.int32),
        mesh=_sc_mesh(),
        scratch_types=[
            pltpu.VMEM((64,), jnp.int32),
            pltpu.VMEM((CHW,), jnp.int32),
        ],
    )
    def k(snd_hbm, out_hbm, hist, pbuf):
        wid = _wid()
        iot = lax.iota(jnp.int32, 16)
        one_i = jnp.ones((16,), jnp.int32)
        zero_i = jnp.zeros((16,), jnp.int32)
        for g in range(4):
            hist[pl.ds(g * 16, 16)] = zero_i
        for ch in range(epw // CHW):
            pltpu.sync_copy(
                snd_hbm.at[pl.ds(wid * epw + ch * CHW, CHW)], pbuf)

            def vec_body(i, c):
                sv = pbuf[pl.ds(i * 16, 16)]
                for j in range(16):
                    bb = _bucket(sv[j])
                    oh = jnp.where(iot == (bb & 15), one_i, zero_i)
                    plsc.addupdate(hist.at[pl.ds((bb >> 4) * 16, 16)], oh)
                return c

            lax.fori_loop(0, CHW // 16, vec_body, 0)
        pltpu.sync_copy(hist, out_hbm.at[pl.ds(wid * 64, 64)])

    return k(senders)


def _sc_scatter(senders, receivers, dx, dy, dz, counts):
    e = senders.shape[0]
    epw = e // NW

    @functools.partial(
        pl.kernel,
        out_type=jax.ShapeDtypeStruct(((e + NBKT * NW * 8 + 16) * 16,),
                                      jnp.float32),
        mesh=_sc_mesh(),
        scratch_types=[
            pltpu.VMEM((NW * 64,), jnp.int32),
            pltpu.VMEM((CHW,), jnp.int32),
            pltpu.VMEM((CHW,), jnp.int32),
            pltpu.VMEM((CHW,), jnp.float32),
            pltpu.VMEM((CHW,), jnp.float32),
            pltpu.VMEM((CHW,), jnp.float32),
            pltpu.VMEM((NBKT * 64 * 16,), jnp.float32),
            pltpu.SMEM((192,), jnp.int32),
            pltpu.SemaphoreType.DMA,
        ],
    )
    def k(snd_hbm, rcv_hbm, dx_hbm, dy_hbm, dz_hbm, cnt_hbm, out_hbm,
          cbuf, sbuf, rbuf, dbx, dby, dbz, stage, smem, sem):
        wid = _wid()
        iot = lax.iota(jnp.int32, 16)
        zero_i = jnp.zeros((16,), jnp.int32)
        zerof = jnp.zeros((16,), jnp.float32)
        lanes = [(iot == kk) for kk in range(7)]
        onef = jnp.where(lanes[6], jnp.full((16,), 1.0), zerof)

        pltpu.sync_copy(cnt_hbm, cbuf)
        T8 = [zero_i] * 4
        PS8 = [zero_i] * 4
        for t in range(NW):
            before = t < wid
            for g in range(4):
                r = cbuf[pl.ds(t * 64 + g * 16, 16)]
                rc = ((r + 7) >> 3) << 3
                T8[g] = T8[g] + rc
                PS8[g] = PS8[g] + jnp.where(before, rc, zero_i)
        s_run8 = jnp.int32(0)
        for b in range(NBKT):
            g, l = b >> 4, b & 15
            smem[b] = s_run8 + (PS8[g][l] >> 3)
            smem[64 + b] = jnp.int32(0)
            s_run8 = s_run8 + (T8[g][l] >> 3)
        smem[190] = jnp.int32(0)

        def drain_flush():
            pltpu.make_async_copy(
                stage.at[pl.ds(0, 128)], out_hbm.at[pl.ds(0, 128)],
                sem).wait()

        def flush8(bb, fbn):
            w = smem[190]

            @pl.when(w >= 6)
            def _():
                drain_flush()

            smem[190] = jnp.where(w >= 6, w, w + 1)
            cur8 = smem[bb]
            slot = ((fbn - 8) & 63) >> 3
            pltpu.async_copy(
                stage.at[pl.ds((bb * 8 + slot) * 128, 128)],
                out_hbm.at[pl.ds(cur8 * 128, 128)], sem)
            smem[bb] = cur8 + 1

        for ch in range(epw // CHW):
            base_c = wid * epw + ch * CHW
            pltpu.sync_copy(snd_hbm.at[pl.ds(base_c, CHW)], sbuf)
            pltpu.sync_copy(rcv_hbm.at[pl.ds(base_c, CHW)], rbuf)
            pltpu.sync_copy(dx_hbm.at[pl.ds(base_c, CHW)], dbx)
            pltpu.sync_copy(dy_hbm.at[pl.ds(base_c, CHW)], dby)
            pltpu.sync_copy(dz_hbm.at[pl.ds(base_c, CHW)], dbz)

            def vec_body(i, c):
                sv = sbuf[pl.ds(i * 16, 16)]
                rv = rbuf[pl.ds(i * 16, 16)]
                xv = dbx[pl.ds(i * 16, 16)]
                yv = dby[pl.ds(i * 16, 16)]
                zv = dbz[pl.ds(i * 16, 16)]
                svf = sv.astype(jnp.float32)
                rvf = rv.astype(jnp.float32)
                for j in range(16):
                    s = sv[j]
                    bb = _bucket(s)
                    eid = (base_c + i * 16 + j).astype(jnp.float32)
                    rec = jnp.where(lanes[0], jnp.full((16,), svf[j]), zerof)
                    rec = jnp.where(lanes[1], jnp.full((16,), rvf[j]), rec)
                    rec = jnp.where(lanes[2], jnp.full((16,), xv[j]), rec)
                    rec = jnp.where(lanes[3], jnp.full((16,), yv[j]), rec)
                    rec = jnp.where(lanes[4], jnp.full((16,), zv[j]), rec)
                    rec = jnp.where(lanes[5], jnp.full((16,), eid), rec)
                    rec = jnp.where(lanes[6], onef, rec)
                    fb = smem[64 + bb]
                    stage[pl.ds((bb * 64 + fb) * 16, 16)] = rec
                    fbn = fb + 1
                    smem[64 + bb] = fbn & 63

                    @pl.when((fbn & 7) == 0)
                    def _(bb=bb, fbn=fbn):
                        flush8(bb, fbn)
                return c

            lax.fori_loop(0, CHW // 16, vec_body, 0)

        def tail(b, c):
            fb = smem[64 + b]
            rem = fb & 7

            @pl.when(rem > 0)
            def _():
                basef = (b * NB).astype(jnp.float32)
                pad = jnp.where(lanes[0], jnp.full((16,), basef), zerof)
                for kk in range(7):
                    @pl.when(kk < 8 - rem)
                    def _(kk=kk):
                        stage[pl.ds((b * 64 + fb + kk) * 16, 16)] = pad
                flush8(b, ((fb >> 3) << 3) + 8)
            return c

        lax.fori_loop(0, NBKT, tail, 0)

        def draini(i, c):
            drain_flush()
            return c

        lax.fori_loop(0, smem[190], draini, 0)

    return k(senders, receivers, dx, dy, dz, counts)


def _sc_main(sorted2d, counts, x2d, v2d, w2d):
    e = w2d.shape[0]
    n2 = x2d.shape[0]

    @functools.partial(
        pl.kernel,
        out_type=jax.ShapeDtypeStruct((NOUT, 4 * H), jnp.float32),
        mesh=_sc_mesh(),
        scratch_types=[
            pltpu.VMEM((NB, 4 * H), jnp.float32),
            pltpu.VMEM((NW * 64,), jnp.int32),
            pltpu.VMEM((256,), jnp.float32),
            pltpu.VMEM((16, H3), jnp.float32),
            pltpu.VMEM((16, H3), jnp.float32),
            pltpu.VMEM((16, H3), jnp.float32),
            pltpu.SMEM((128,), jnp.int32),
        ],
    )
    def k(rec_hbm, cnt_hbm, x_hbm, v_hbm, w_hbm, out_hbm,
          acc, cbuf, recbuf, wg, xg, vg, smem):
        wid = _wid()
        iot = lax.iota(jnp.int32, 16)
        zero16 = jnp.zeros((16,), jnp.float32)
        zero_i = jnp.zeros((16,), jnp.int32)

        pltpu.sync_copy(cnt_hbm, cbuf)
        T8 = [zero_i] * 4
        for t in range(NW):
            for g in range(4):
                r = cbuf[pl.ds(t * 64 + g * 16, 16)]
                T8[g] = T8[g] + (((r + 7) >> 3) << 3)
        s_run8 = jnp.int32(0)
        for b in range(NBKT):
            smem[b] = s_run8
            smem[64 + b] = T8[b >> 4][b & 15]
            s_run8 = s_run8 + (T8[b >> 4][b & 15] >> 3)

        for r in range(ROUNDS):
            b = r * NW + wid

            @pl.when(b < NBKT)
            def _():
                base = b * NB

                def zrow(i, c):
                    for cc in range(4 * H // 16):
                        acc[i, pl.ds(cc * 16, 16)] = zero16
                    return c

                lax.fori_loop(0, NB, zrow, 0)
                lo8 = smem[b]
                cnt = smem[64 + b]
                nch = (cnt + 15) >> 4

                def chunk_body(ch, c):
                    cbase = (lo8 + ch * 2) * 128
                    pltpu.sync_copy(
                        rec_hbm.at[pl.ds(cbase, 256)], recbuf)
                    idxe = zero_i
                    idxr = zero_i
                    for j in range(16):
                        recj = recbuf[pl.ds(j * 16, 16)]
                        ei = jnp.full((16,), jnp.int32(recj[5]), jnp.int32)
                        ri = jnp.full((16,), jnp.int32(recj[1]), jnp.int32)
                        idxe = jnp.where(iot == j, ei, idxe)
                        idxr = jnp.where(iot == j, ri, idxr)
                    idxe = jnp.clip(idxe, 0, e - 1)
                    idxr = jnp.clip(idxr, 0, n2 - 1)
                    pltpu.sync_copy(w_hbm.at[idxe], wg)
                    pltpu.sync_copy(x_hbm.at[idxr], xg)
                    pltpu.sync_copy(v_hbm.at[idxr], vg)
                    nvalid = jnp.minimum(cnt - ch * 16, 16)

                    def compute(j, c0):
                        recj = recbuf[pl.ds(j * 16, 16)]
                        row = jnp.int32(recj[0]) - base
                        sc = recj[6]
                        scv = jnp.full((16,), sc, jnp.float32)
                        dvecs = [jnp.full((16,), recj[2 + kk] * sc,
                                          jnp.float32)
                                 for kk in range(3)]
                        for c in range(8):
                            co = c * 16
                            wv0 = wg[j, pl.ds(co, 16)]
                            wv1 = wg[j, pl.ds(H + co, 16)]
                            wv2 = wg[j, pl.ds(2 * H + co, 16)]
                            xv0 = xg[j, pl.ds(co, 16)]
                            xv1 = xg[j, pl.ds(H + co, 16)]
                            xv2 = xg[j, pl.ds(2 * H + co, 16)]
                            dv1 = wv1 * xv1
                            dv2 = wv2 * xv2 * scv
                            plsc.addupdate(
                                acc.at[row, pl.ds(co, 16)], wv0 * xv0 * scv)
                            for kk in range(3):
                                vj = vg[j, pl.ds(kk * H + co, 16)]
                                plsc.addupdate(
                                    acc.at[row, pl.ds(H + kk * H + co, 16)],
                                    dv1 * dvecs[kk] + dv2 * vj)
                        return c0

                    lax.fori_loop(0, nvalid, compute, 0)
                    return c

                lax.fori_loop(0, nch, chunk_body, 0)
                pltpu.sync_copy(acc, out_hbm.at[pl.ds(base, NB)])

    return k(sorted2d, counts, x2d, v2d, w2d)


def _update(s2d, v2d, dsum, dvsum, W_vmix, W_mix1, b_mix1, W_mix2, b_mix2):
    n = s2d.shape[0]
    bs = 400

    def body(s_ref, v_ref, ds_ref, dv_ref, wv_ref, w1_ref, b1_ref, w2_ref,
             b2_ref, so_ref, vo_ref):
        s1 = s_ref[...] + _clip(ds_ref[...])
        v1 = v_ref[...] + _clip(dv_ref[...])
        wv = wv_ref[...]
        v1k = [v1[:, kk * H:(kk + 1) * H] for kk in range(3)]
        vm = [jnp.dot(vk, wv, preferred_element_type=jnp.float32) for vk in v1k]
        v_l = [m[:, :H] for m in vm]
        v_r = [m[:, H:] for m in vm]
        nsq = v_r[0] * v_r[0] + v_r[1] * v_r[1] + v_r[2] * v_r[2]
        v_norm = jnp.sqrt(nsq + EPS)
        w1 = w1_ref[...]
        h = (jnp.dot(s1, w1[:H, :], preferred_element_type=jnp.float32)
             + jnp.dot(v_norm, w1[H:, :], preferred_element_type=jnp.float32)
             + b1_ref[...])
        h = h * jax.nn.sigmoid(h)
        m = jnp.dot(h, w2_ref[...], preferred_element_type=jnp.float32)
        m = m + b2_ref[...]
        ds2 = m[:, :H]
        dvu_g = m[:, H:2 * H]
        dsv_g = m[:, 2 * H:]
        dot_rl = v_r[0] * v_l[0] + v_r[1] * v_l[1] + v_r[2] * v_l[2]
        so_ref[...] = s1 + _clip(ds2 + dsv_g * dot_rl)
        vo_ref[...] = jnp.concatenate(
            [v1k[kk] + _clip(v_l[kk] * dvu_g) for kk in range(3)], axis=1)

    return pl.pallas_call(
        body,
        grid=(n // bs,),
        in_specs=[
            pl.BlockSpec((bs, H), lambda i: (i, 0)),
            pl.BlockSpec((bs, H3), lambda i: (i, 0)),
            pl.BlockSpec((bs, H), lambda i: (i, 0)),
            pl.BlockSpec((bs, H3), lambda i: (i, 0)),
            pl.BlockSpec((H, 2 * H), lambda i: (0, 0)),
            pl.BlockSpec((2 * H, H), lambda i: (0, 0)),
            pl.BlockSpec((1, H), lambda i: (0, 0)),
            pl.BlockSpec((H, H3), lambda i: (0, 0)),
            pl.BlockSpec((1, H3), lambda i: (0, 0)),
        ],
        out_specs=[
            pl.BlockSpec((bs, H), lambda i: (i, 0)),
            pl.BlockSpec((bs, H3), lambda i: (i, 0)),
        ],
        out_shape=[
            jax.ShapeDtypeStruct((n, H), jnp.float32),
            jax.ShapeDtypeStruct((n, H3), jnp.float32),
        ],
    )(s2d, v2d, dsum, dvsum, W_vmix, W_mix1, b_mix1.reshape(1, H), W_mix2,
      b_mix2.reshape(1, H3))


def kernel(s, v, dir_ij, Wij, senders, receivers, W_int1, b_int1, W_int2,
           b_int2, W_vmix, W_mix1, b_mix1, W_mix2, b_mix2):
    n = s.shape[0]
    e = senders.shape[0]
    s2d = s.reshape(n, H)
    v2d = v.reshape(n, H3)
    w2d = Wij.reshape(e, H3)

    x2d = _mlp1(s2d, W_int1, b_int1, W_int2, b_int2)

    counts = _sc_hist(senders)
    sorted2d = _sc_scatter(senders, receivers, dir_ij[:, 0], dir_ij[:, 1],
                           dir_ij[:, 2], counts)
    agg = _sc_main(sorted2d, counts, x2d, v2d, w2d)
    dsum = agg[:n, :H]
    dvsum = agg[:n, H:]

    s_out, v_out = _update(s2d, v2d, dsum, dvsum, W_vmix, W_mix1, b_mix1,
                           W_mix2, b_mix2)
    return (s_out.reshape(n, 1, H), v_out.reshape(n, 3, H))


# speedup vs baseline: 4.9668x; 1.2001x over previous
import functools

import jax
import jax.numpy as jnp
from jax import lax
from jax.experimental import pallas as pl
from jax.experimental.pallas import tpu as pltpu
from jax.experimental.pallas import tpu_sc as plsc

H = 128
H3 = 3 * H
EPS = 1e-08

NW = 32
NB = 160
NBKT = 63
ROUNDS = 2
NOUT = NBKT * NB
CHW = 2000


def _clip(x):
    return jnp.clip(x, -100.0, 100.0)


def _bucket(snd):
    return ((snd >> 5) * 13108) >> 16


def _mlp1(s2d, W1, b1, W2, b2):
    n = s2d.shape[0]
    bs = 400

    def body(s_ref, w1_ref, b1_ref, w2_ref, b2_ref, o_ref):
        h = jnp.dot(s_ref[...], w1_ref[...], preferred_element_type=jnp.float32)
        h = h + b1_ref[...]
        h = h * jax.nn.sigmoid(h)
        o = jnp.dot(h, w2_ref[...], preferred_element_type=jnp.float32)
        o_ref[...] = o + b2_ref[...]

    return pl.pallas_call(
        body,
        grid=(n // bs,),
        in_specs=[
            pl.BlockSpec((bs, H), lambda i: (i, 0)),
            pl.BlockSpec((H, H), lambda i: (0, 0)),
            pl.BlockSpec((1, H), lambda i: (0, 0)),
            pl.BlockSpec((H, H3), lambda i: (0, 0)),
            pl.BlockSpec((1, H3), lambda i: (0, 0)),
        ],
        out_specs=pl.BlockSpec((bs, H3), lambda i: (i, 0)),
        out_shape=jax.ShapeDtypeStruct((n, H3), jnp.float32),
    )(s2d, W1, b1.reshape(1, H), W2, b2.reshape(1, H3))


def _sc_mesh():
    return plsc.VectorSubcoreMesh(core_axis_name="c", subcore_axis_name="s")


def _wid():
    return lax.axis_index("s") * 2 + lax.axis_index("c")


def _sc_hist(senders):
    e = senders.shape[0]
    epw = e // NW

    @functools.partial(
        pl.kernel,
        out_type=jax.ShapeDtypeStruct((NW * 64,), jnp.int32),
        mesh=_sc_mesh(),
        scratch_types=[
            pltpu.VMEM((64,), jnp.int32),
            pltpu.VMEM((CHW,), jnp.int32),
        ],
    )
    def k(snd_hbm, out_hbm, hist, pbuf):
        wid = _wid()
        iot = lax.iota(jnp.int32, 16)
        one_i = jnp.ones((16,), jnp.int32)
        zero_i = jnp.zeros((16,), jnp.int32)
        for g in range(4):
            hist[pl.ds(g * 16, 16)] = zero_i
        for ch in range(epw // CHW):
            pltpu.sync_copy(
                snd_hbm.at[pl.ds(wid * epw + ch * CHW, CHW)], pbuf)

            def vec_body(i, c):
                sv = pbuf[pl.ds(i * 16, 16)]
                for j in range(16):
                    bb = _bucket(sv[j])
                    oh = jnp.where(iot == (bb & 15), one_i, zero_i)
                    plsc.addupdate(hist.at[pl.ds((bb >> 4) * 16, 16)], oh)
                return c

            lax.fori_loop(0, CHW // 16, vec_body, 0)
        pltpu.sync_copy(hist, out_hbm.at[pl.ds(wid * 64, 64)])

    return k(senders)


def _sc_scatter(senders, receivers, dx, dy, dz, counts):
    e = senders.shape[0]
    epw = e // NW

    @functools.partial(
        pl.kernel,
        out_type=jax.ShapeDtypeStruct(((e + NBKT * NW * 8 + 16) * 16,),
                                      jnp.float32),
        mesh=_sc_mesh(),
        scratch_types=[
            pltpu.VMEM((NW * 64,), jnp.int32),
            pltpu.VMEM((CHW,), jnp.int32),
            pltpu.VMEM((CHW,), jnp.int32),
            pltpu.VMEM((CHW,), jnp.float32),
            pltpu.VMEM((CHW,), jnp.float32),
            pltpu.VMEM((CHW,), jnp.float32),
            pltpu.VMEM((NBKT * 64 * 16,), jnp.float32),
            pltpu.SMEM((192,), jnp.int32),
            pltpu.SemaphoreType.DMA,
        ],
    )
    def k(snd_hbm, rcv_hbm, dx_hbm, dy_hbm, dz_hbm, cnt_hbm, out_hbm,
          cbuf, sbuf, rbuf, dbx, dby, dbz, stage, smem, sem):
        wid = _wid()
        iot = lax.iota(jnp.int32, 16)
        zero_i = jnp.zeros((16,), jnp.int32)
        zerof = jnp.zeros((16,), jnp.float32)
        lanes = [(iot == kk) for kk in range(7)]
        onef = jnp.where(lanes[6], jnp.full((16,), 1.0), zerof)

        pltpu.sync_copy(cnt_hbm, cbuf)
        T8 = [zero_i] * 4
        PS8 = [zero_i] * 4
        for t in range(NW):
            before = t < wid
            for g in range(4):
                r = cbuf[pl.ds(t * 64 + g * 16, 16)]
                rc = ((r + 7) >> 3) << 3
                T8[g] = T8[g] + rc
                PS8[g] = PS8[g] + jnp.where(before, rc, zero_i)
        s_run8 = jnp.int32(0)
        for b in range(NBKT):
            g, l = b >> 4, b & 15
            smem[b] = s_run8 + (PS8[g][l] >> 3)
            smem[64 + b] = jnp.int32(0)
            s_run8 = s_run8 + (T8[g][l] >> 3)
        smem[190] = jnp.int32(0)

        def drain_flush():
            pltpu.make_async_copy(
                stage.at[pl.ds(0, 128)], out_hbm.at[pl.ds(0, 128)],
                sem).wait()

        def flush8(bb, fbn):
            w = smem[190]

            @pl.when(w >= 6)
            def _():
                drain_flush()

            smem[190] = jnp.where(w >= 6, w, w + 1)
            cur8 = smem[bb]
            slot = ((fbn - 8) & 63) >> 3
            pltpu.async_copy(
                stage.at[pl.ds((bb * 8 + slot) * 128, 128)],
                out_hbm.at[pl.ds(cur8 * 128, 128)], sem)
            smem[bb] = cur8 + 1

        for ch in range(epw // CHW):
            base_c = wid * epw + ch * CHW
            pltpu.sync_copy(snd_hbm.at[pl.ds(base_c, CHW)], sbuf)
            pltpu.sync_copy(rcv_hbm.at[pl.ds(base_c, CHW)], rbuf)
            pltpu.sync_copy(dx_hbm.at[pl.ds(base_c, CHW)], dbx)
            pltpu.sync_copy(dy_hbm.at[pl.ds(base_c, CHW)], dby)
            pltpu.sync_copy(dz_hbm.at[pl.ds(base_c, CHW)], dbz)

            def vec_body(i, c):
                sv = sbuf[pl.ds(i * 16, 16)]
                rv = rbuf[pl.ds(i * 16, 16)]
                xv = dbx[pl.ds(i * 16, 16)]
                yv = dby[pl.ds(i * 16, 16)]
                zv = dbz[pl.ds(i * 16, 16)]
                svf = sv.astype(jnp.float32)
                rvf = rv.astype(jnp.float32)
                for j in range(16):
                    s = sv[j]
                    bb = _bucket(s)
                    eid = (base_c + i * 16 + j).astype(jnp.float32)
                    rec = jnp.where(lanes[0], jnp.full((16,), svf[j]), zerof)
                    rec = jnp.where(lanes[1], jnp.full((16,), rvf[j]), rec)
                    rec = jnp.where(lanes[2], jnp.full((16,), xv[j]), rec)
                    rec = jnp.where(lanes[3], jnp.full((16,), yv[j]), rec)
                    rec = jnp.where(lanes[4], jnp.full((16,), zv[j]), rec)
                    rec = jnp.where(lanes[5], jnp.full((16,), eid), rec)
                    rec = jnp.where(lanes[6], onef, rec)
                    fb = smem[64 + bb]
                    stage[pl.ds((bb * 64 + fb) * 16, 16)] = rec
                    fbn = fb + 1
                    smem[64 + bb] = fbn & 63

                    @pl.when((fbn & 7) == 0)
                    def _(bb=bb, fbn=fbn):
                        flush8(bb, fbn)
                return c

            lax.fori_loop(0, CHW // 16, vec_body, 0)

        def tail(b, c):
            fb = smem[64 + b]
            rem = fb & 7

            @pl.when(rem > 0)
            def _():
                basef = (b * NB).astype(jnp.float32)
                pad = jnp.where(lanes[0], jnp.full((16,), basef), zerof)
                for kk in range(7):
                    @pl.when(kk < 8 - rem)
                    def _(kk=kk):
                        stage[pl.ds((b * 64 + fb + kk) * 16, 16)] = pad
                flush8(b, ((fb >> 3) << 3) + 8)
            return c

        lax.fori_loop(0, NBKT, tail, 0)

        def draini(i, c):
            drain_flush()
            return c

        lax.fori_loop(0, smem[190], draini, 0)

    return k(senders, receivers, dx, dy, dz, counts)


def _sc_main(sorted2d, counts, x2d, v2d, w2d):
    e = w2d.shape[0]
    n2 = x2d.shape[0]

    @functools.partial(
        pl.kernel,
        out_type=jax.ShapeDtypeStruct((NOUT, 4 * H), jnp.float32),
        mesh=_sc_mesh(),
        scratch_types=[
            pltpu.VMEM((NB, 4 * H), jnp.float32),
            pltpu.VMEM((NW * 64,), jnp.int32),
            pltpu.VMEM((256,), jnp.float32),
            pltpu.VMEM((16, H3), jnp.float32),
            pltpu.VMEM((16, H3), jnp.float32),
            pltpu.VMEM((16, H3), jnp.float32),
            pltpu.SMEM((128,), jnp.int32),
            pltpu.SemaphoreType.DMA,
        ],
    )
    def k(rec_hbm, cnt_hbm, x_hbm, v_hbm, w_hbm, out_hbm,
          acc, cbuf, recbuf, wg, xg, vg, smem, gsem):
        wid = _wid()
        iot = lax.iota(jnp.int32, 16)
        zero16 = jnp.zeros((16,), jnp.float32)
        zero_i = jnp.zeros((16,), jnp.int32)

        pltpu.sync_copy(cnt_hbm, cbuf)
        T8 = [zero_i] * 4
        for t in range(NW):
            for g in range(4):
                r = cbuf[pl.ds(t * 64 + g * 16, 16)]
                T8[g] = T8[g] + (((r + 7) >> 3) << 3)
        s_run8 = jnp.int32(0)
        for b in range(NBKT):
            smem[b] = s_run8
            smem[64 + b] = T8[b >> 4][b & 15]
            s_run8 = s_run8 + (T8[b >> 4][b & 15] >> 3)

        for r in range(ROUNDS):
            b = r * NW + wid

            @pl.when(b < NBKT)
            def _():
                base = b * NB

                def zrow(i, c):
                    for cc in range(4 * H // 16):
                        acc[i, pl.ds(cc * 16, 16)] = zero16
                    return c

                lax.fori_loop(0, NB, zrow, 0)
                lo8 = smem[b]
                cnt = smem[64 + b]
                nch = (cnt + 15) >> 4

                def chunk_body(ch, c):
                    cbase = (lo8 + ch * 2) * 128
                    pltpu.sync_copy(
                        rec_hbm.at[pl.ds(cbase, 256)], recbuf)
                    idxe = zero_i
                    idxr = zero_i
                    for j in range(16):
                        recj = recbuf[pl.ds(j * 16, 16)]
                        ei = jnp.full((16,), jnp.int32(recj[5]), jnp.int32)
                        ri = jnp.full((16,), jnp.int32(recj[1]), jnp.int32)
                        idxe = jnp.where(iot == j, ei, idxe)
                        idxr = jnp.where(iot == j, ri, idxr)
                    idxe = jnp.clip(idxe, 0, e - 1)
                    idxr = jnp.clip(idxr, 0, n2 - 1)
                    pltpu.async_copy(w_hbm.at[idxe], wg, gsem)
                    pltpu.async_copy(x_hbm.at[idxr], xg, gsem)
                    pltpu.async_copy(v_hbm.at[idxr], vg, gsem)
                    for _ in range(3):
                        pltpu.make_async_copy(
                            x_hbm.at[idxr], xg, gsem).wait()
                    nvalid = jnp.minimum(cnt - ch * 16, 16)

                    def compute(j, c0):
                        recj = recbuf[pl.ds(j * 16, 16)]
                        row = jnp.int32(recj[0]) - base
                        sc = recj[6]
                        scv = jnp.full((16,), sc, jnp.float32)
                        dvecs = [jnp.full((16,), recj[2 + kk] * sc,
                                          jnp.float32)
                                 for kk in range(3)]
                        for c in range(8):
                            co = c * 16
                            wv0 = wg[j, pl.ds(co, 16)]
                            wv1 = wg[j, pl.ds(H + co, 16)]
                            wv2 = wg[j, pl.ds(2 * H + co, 16)]
                            xv0 = xg[j, pl.ds(co, 16)]
                            xv1 = xg[j, pl.ds(H + co, 16)]
                            xv2 = xg[j, pl.ds(2 * H + co, 16)]
                            dv1 = wv1 * xv1
                            dv2 = wv2 * xv2 * scv
                            plsc.addupdate(
                                acc.at[row, pl.ds(co, 16)], wv0 * xv0 * scv)
                            for kk in range(3):
                                vj = vg[j, pl.ds(kk * H + co, 16)]
                                plsc.addupdate(
                                    acc.at[row, pl.ds(H + kk * H + co, 16)],
                                    dv1 * dvecs[kk] + dv2 * vj)
                        return c0

                    lax.fori_loop(0, nvalid, compute, 0)
                    return c

                lax.fori_loop(0, nch, chunk_body, 0)
                pltpu.sync_copy(acc, out_hbm.at[pl.ds(base, NB)])

    return k(sorted2d, counts, x2d, v2d, w2d)


def _update(s2d, v2d, dsum, dvsum, W_vmix, W_mix1, b_mix1, W_mix2, b_mix2):
    n = s2d.shape[0]
    bs = 400

    def body(s_ref, v_ref, ds_ref, dv_ref, wv_ref, w1_ref, b1_ref, w2_ref,
             b2_ref, so_ref, vo_ref):
        s1 = s_ref[...] + _clip(ds_ref[...])
        v1 = v_ref[...] + _clip(dv_ref[...])
        wv = wv_ref[...]
        v1k = [v1[:, kk * H:(kk + 1) * H] for kk in range(3)]
        vm = [jnp.dot(vk, wv, preferred_element_type=jnp.float32) for vk in v1k]
        v_l = [m[:, :H] for m in vm]
        v_r = [m[:, H:] for m in vm]
        nsq = v_r[0] * v_r[0] + v_r[1] * v_r[1] + v_r[2] * v_r[2]
        v_norm = jnp.sqrt(nsq + EPS)
        w1 = w1_ref[...]
        h = (jnp.dot(s1, w1[:H, :], preferred_element_type=jnp.float32)
             + jnp.dot(v_norm, w1[H:, :], preferred_element_type=jnp.float32)
             + b1_ref[...])
        h = h * jax.nn.sigmoid(h)
        m = jnp.dot(h, w2_ref[...], preferred_element_type=jnp.float32)
        m = m + b2_ref[...]
        ds2 = m[:, :H]
        dvu_g = m[:, H:2 * H]
        dsv_g = m[:, 2 * H:]
        dot_rl = v_r[0] * v_l[0] + v_r[1] * v_l[1] + v_r[2] * v_l[2]
        so_ref[...] = s1 + _clip(ds2 + dsv_g * dot_rl)
        vo_ref[...] = jnp.concatenate(
            [v1k[kk] + _clip(v_l[kk] * dvu_g) for kk in range(3)], axis=1)

    return pl.pallas_call(
        body,
        grid=(n // bs,),
        in_specs=[
            pl.BlockSpec((bs, H), lambda i: (i, 0)),
            pl.BlockSpec((bs, H3), lambda i: (i, 0)),
            pl.BlockSpec((bs, H), lambda i: (i, 0)),
            pl.BlockSpec((bs, H3), lambda i: (i, 0)),
            pl.BlockSpec((H, 2 * H), lambda i: (0, 0)),
            pl.BlockSpec((2 * H, H), lambda i: (0, 0)),
            pl.BlockSpec((1, H), lambda i: (0, 0)),
            pl.BlockSpec((H, H3), lambda i: (0, 0)),
            pl.BlockSpec((1, H3), lambda i: (0, 0)),
        ],
        out_specs=[
            pl.BlockSpec((bs, H), lambda i: (i, 0)),
            pl.BlockSpec((bs, H3), lambda i: (i, 0)),
        ],
        out_shape=[
            jax.ShapeDtypeStruct((n, H), jnp.float32),
            jax.ShapeDtypeStruct((n, H3), jnp.float32),
        ],
    )(s2d, v2d, dsum, dvsum, W_vmix, W_mix1, b_mix1.reshape(1, H), W_mix2,
      b_mix2.reshape(1, H3))


def kernel(s, v, dir_ij, Wij, senders, receivers, W_int1, b_int1, W_int2,
           b_int2, W_vmix, W_mix1, b_mix1, W_mix2, b_mix2):
    n = s.shape[0]
    e = senders.shape[0]
    s2d = s.reshape(n, H)
    v2d = v.reshape(n, H3)
    w2d = Wij.reshape(e, H3)

    x2d = _mlp1(s2d, W_int1, b_int1, W_int2, b_int2)

    counts = _sc_hist(senders)
    sorted2d = _sc_scatter(senders, receivers, dir_ij[:, 0], dir_ij[:, 1],
                           dir_ij[:, 2], counts)
    agg = _sc_main(sorted2d, counts, x2d, v2d, w2d)
    dsum = agg[:n, :H]
    dvsum = agg[:n, H:]

    s_out, v_out = _update(s2d, v2d, dsum, dvsum, W_vmix, W_mix1, b_mix1,
                           W_mix2, b_mix2)
    return (s_out.reshape(n, 1, H), v_out.reshape(n, 3, H))


# speedup vs baseline: 6.3266x; 1.2738x over previous
import functools

import jax
import jax.numpy as jnp
from jax import lax
from jax.experimental import pallas as pl
from jax.experimental.pallas import tpu as pltpu
from jax.experimental.pallas import tpu_sc as plsc

H = 128
H3 = 3 * H
EPS = 1e-08

NW = 32
NB = 160
NBKT = 63
ROUNDS = 2
NOUT = NBKT * NB
CHW = 2000


def _clip(x):
    return jnp.clip(x, -100.0, 100.0)


def _bucket(snd):
    return ((snd >> 5) * 13108) >> 16


def _mlp1(s2d, W1, b1, W2, b2):
    n = s2d.shape[0]
    bs = 400

    def body(s_ref, w1_ref, b1_ref, w2_ref, b2_ref, o_ref):
        h = jnp.dot(s_ref[...], w1_ref[...], preferred_element_type=jnp.float32)
        h = h + b1_ref[...]
        h = h * jax.nn.sigmoid(h)
        o = jnp.dot(h, w2_ref[...], preferred_element_type=jnp.float32)
        o_ref[...] = o + b2_ref[...]

    return pl.pallas_call(
        body,
        grid=(n // bs,),
        in_specs=[
            pl.BlockSpec((bs, H), lambda i: (i, 0)),
            pl.BlockSpec((H, H), lambda i: (0, 0)),
            pl.BlockSpec((1, H), lambda i: (0, 0)),
            pl.BlockSpec((H, H3), lambda i: (0, 0)),
            pl.BlockSpec((1, H3), lambda i: (0, 0)),
        ],
        out_specs=pl.BlockSpec((bs, H3), lambda i: (i, 0)),
        out_shape=jax.ShapeDtypeStruct((n, H3), jnp.float32),
    )(s2d, W1, b1.reshape(1, H), W2, b2.reshape(1, H3))


def _sc_mesh():
    return plsc.VectorSubcoreMesh(core_axis_name="c", subcore_axis_name="s")


def _wid():
    return lax.axis_index("s") * 2 + lax.axis_index("c")


def _sc_hist(senders):
    e = senders.shape[0]
    epw = e // NW

    @functools.partial(
        pl.kernel,
        out_type=jax.ShapeDtypeStruct((NW * 64,), jnp.int32),
        mesh=_sc_mesh(),
        scratch_types=[
            pltpu.VMEM((64,), jnp.int32),
            pltpu.VMEM((CHW,), jnp.int32),
        ],
    )
    def k(snd_hbm, out_hbm, hist, pbuf):
        wid = _wid()
        iot = lax.iota(jnp.int32, 16)
        one_i = jnp.ones((16,), jnp.int32)
        zero_i = jnp.zeros((16,), jnp.int32)
        for g in range(4):
            hist[pl.ds(g * 16, 16)] = zero_i
        for ch in range(epw // CHW):
            pltpu.sync_copy(
                snd_hbm.at[pl.ds(wid * epw + ch * CHW, CHW)], pbuf)

            def vec_body(i, c):
                sv = pbuf[pl.ds(i * 16, 16)]
                for j in range(16):
                    bb = _bucket(sv[j])
                    oh = jnp.where(iot == (bb & 15), one_i, zero_i)
                    plsc.addupdate(hist.at[pl.ds((bb >> 4) * 16, 16)], oh)
                return c

            lax.fori_loop(0, CHW // 16, vec_body, 0)
        pltpu.sync_copy(hist, out_hbm.at[pl.ds(wid * 64, 64)])

    return k(senders)


def _sc_scatter(senders, receivers, dx, dy, dz, counts):
    e = senders.shape[0]
    epw = e // NW

    @functools.partial(
        pl.kernel,
        out_type=jax.ShapeDtypeStruct(((e + NBKT * NW * 8 + 16) * 16,),
                                      jnp.float32),
        mesh=_sc_mesh(),
        scratch_types=[
            pltpu.VMEM((NW * 64,), jnp.int32),
            pltpu.VMEM((CHW,), jnp.int32),
            pltpu.VMEM((CHW,), jnp.int32),
            pltpu.VMEM((CHW,), jnp.float32),
            pltpu.VMEM((CHW,), jnp.float32),
            pltpu.VMEM((CHW,), jnp.float32),
            pltpu.VMEM((NBKT * 64 * 16,), jnp.float32),
            pltpu.SMEM((192,), jnp.int32),
            pltpu.SemaphoreType.DMA,
        ],
    )
    def k(snd_hbm, rcv_hbm, dx_hbm, dy_hbm, dz_hbm, cnt_hbm, out_hbm,
          cbuf, sbuf, rbuf, dbx, dby, dbz, stage, smem, sem):
        wid = _wid()
        iot = lax.iota(jnp.int32, 16)
        zero_i = jnp.zeros((16,), jnp.int32)
        zerof = jnp.zeros((16,), jnp.float32)
        lanes = [(iot == kk) for kk in range(7)]
        onef = jnp.where(lanes[6], jnp.full((16,), 1.0), zerof)

        pltpu.sync_copy(cnt_hbm, cbuf)
        T8 = [zero_i] * 4
        PS8 = [zero_i] * 4
        for t in range(NW):
            before = t < wid
            for g in range(4):
                r = cbuf[pl.ds(t * 64 + g * 16, 16)]
                rc = ((r + 7) >> 3) << 3
                T8[g] = T8[g] + rc
                PS8[g] = PS8[g] + jnp.where(before, rc, zero_i)
        s_run8 = jnp.int32(0)
        for b in range(NBKT):
            g, l = b >> 4, b & 15
            smem[b] = s_run8 + (PS8[g][l] >> 3)
            smem[64 + b] = jnp.int32(0)
            s_run8 = s_run8 + (T8[g][l] >> 3)
        smem[190] = jnp.int32(0)

        def drain_flush():
            pltpu.make_async_copy(
                stage.at[pl.ds(0, 128)], out_hbm.at[pl.ds(0, 128)],
                sem).wait()

        def flush8(bb, fbn):
            w = smem[190]

            @pl.when(w >= 6)
            def _():
                drain_flush()

            smem[190] = jnp.where(w >= 6, w, w + 1)
            cur8 = smem[bb]
            slot = ((fbn - 8) & 63) >> 3
            pltpu.async_copy(
                stage.at[pl.ds((bb * 8 + slot) * 128, 128)],
                out_hbm.at[pl.ds(cur8 * 128, 128)], sem)
            smem[bb] = cur8 + 1

        for ch in range(epw // CHW):
            base_c = wid * epw + ch * CHW
            pltpu.sync_copy(snd_hbm.at[pl.ds(base_c, CHW)], sbuf)
            pltpu.sync_copy(rcv_hbm.at[pl.ds(base_c, CHW)], rbuf)
            pltpu.sync_copy(dx_hbm.at[pl.ds(base_c, CHW)], dbx)
            pltpu.sync_copy(dy_hbm.at[pl.ds(base_c, CHW)], dby)
            pltpu.sync_copy(dz_hbm.at[pl.ds(base_c, CHW)], dbz)

            def vec_body(i, c):
                sv = sbuf[pl.ds(i * 16, 16)]
                rv = rbuf[pl.ds(i * 16, 16)]
                xv = dbx[pl.ds(i * 16, 16)]
                yv = dby[pl.ds(i * 16, 16)]
                zv = dbz[pl.ds(i * 16, 16)]
                svf = sv.astype(jnp.float32)
                rvf = rv.astype(jnp.float32)
                for j in range(16):
                    s = sv[j]
                    bb = _bucket(s)
                    eid = (base_c + i * 16 + j).astype(jnp.float32)
                    rec = jnp.where(lanes[0], jnp.full((16,), svf[j]), zerof)
                    rec = jnp.where(lanes[1], jnp.full((16,), rvf[j]), rec)
                    rec = jnp.where(lanes[2], jnp.full((16,), xv[j]), rec)
                    rec = jnp.where(lanes[3], jnp.full((16,), yv[j]), rec)
                    rec = jnp.where(lanes[4], jnp.full((16,), zv[j]), rec)
                    rec = jnp.where(lanes[5], jnp.full((16,), eid), rec)
                    rec = jnp.where(lanes[6], onef, rec)
                    fb = smem[64 + bb]
                    stage[pl.ds((bb * 64 + fb) * 16, 16)] = rec
                    fbn = fb + 1
                    smem[64 + bb] = fbn & 63

                    @pl.when((fbn & 7) == 0)
                    def _(bb=bb, fbn=fbn):
                        flush8(bb, fbn)
                return c

            lax.fori_loop(0, CHW // 16, vec_body, 0)

        def tail(b, c):
            fb = smem[64 + b]
            rem = fb & 7

            @pl.when(rem > 0)
            def _():
                basef = (b * NB).astype(jnp.float32)
                pad = jnp.where(lanes[0], jnp.full((16,), basef), zerof)
                for kk in range(7):
                    @pl.when(kk < 8 - rem)
                    def _(kk=kk):
                        stage[pl.ds((b * 64 + fb + kk) * 16, 16)] = pad
                flush8(b, ((fb >> 3) << 3) + 8)
            return c

        lax.fori_loop(0, NBKT, tail, 0)

        def draini(i, c):
            drain_flush()
            return c

        lax.fori_loop(0, smem[190], draini, 0)

    return k(senders, receivers, dx, dy, dz, counts)


def _sc_main(sorted2d, counts, x2d, v2d, w2d):
    e = w2d.shape[0]
    n2 = x2d.shape[0]

    @functools.partial(
        pl.kernel,
        out_type=jax.ShapeDtypeStruct((NOUT, 4 * H), jnp.float32),
        mesh=_sc_mesh(),
        scratch_types=[
            pltpu.VMEM((NB, 4 * H), jnp.float32),
            pltpu.VMEM((NW * 64,), jnp.int32),
            pltpu.VMEM((256,), jnp.float32),
            pltpu.VMEM((256,), jnp.float32),
            pltpu.VMEM((16, H3), jnp.float32),
            pltpu.VMEM((16, H3), jnp.float32),
            pltpu.VMEM((16, H3), jnp.float32),
            pltpu.VMEM((16, H3), jnp.float32),
            pltpu.VMEM((16, H3), jnp.float32),
            pltpu.VMEM((16, H3), jnp.float32),
            pltpu.SMEM((128,), jnp.int32),
            pltpu.SemaphoreType.DMA,
            pltpu.SemaphoreType.DMA,
        ],
    )
    def k(rec_hbm, cnt_hbm, x_hbm, v_hbm, w_hbm, out_hbm,
          acc, cbuf, rb0, rb1, wg0, wg1, xg0, xg1, vg0, vg1, smem,
          sem0, sem1):
        rbufs = (rb0, rb1)
        wgs = (wg0, wg1)
        xgs = (xg0, xg1)
        vgs = (vg0, vg1)
        sems = (sem0, sem1)
        wid = _wid()
        iot = lax.iota(jnp.int32, 16)
        zero16 = jnp.zeros((16,), jnp.float32)
        zero_i = jnp.zeros((16,), jnp.int32)

        pltpu.sync_copy(cnt_hbm, cbuf)
        T8 = [zero_i] * 4
        for t in range(NW):
            for g in range(4):
                r = cbuf[pl.ds(t * 64 + g * 16, 16)]
                T8[g] = T8[g] + (((r + 7) >> 3) << 3)
        s_run8 = jnp.int32(0)
        for b in range(NBKT):
            smem[b] = s_run8
            smem[64 + b] = T8[b >> 4][b & 15]
            s_run8 = s_run8 + (T8[b >> 4][b & 15] >> 3)

        for r in range(ROUNDS):
            b = r * NW + wid

            @pl.when(b < NBKT)
            def _():
                base = b * NB

                def zrow(i, c):
                    for cc in range(4 * H // 16):
                        acc[i, pl.ds(cc * 16, 16)] = zero16
                    return c

                lax.fori_loop(0, NB, zrow, 0)
                lo8 = smem[b]
                cnt = smem[64 + b]
                nch = (cnt + 15) >> 4

                def fire(s, ch2):
                    cbase = (lo8 + ch2 * 2) * 128
                    pltpu.sync_copy(
                        rec_hbm.at[pl.ds(cbase, 256)], rbufs[s])
                    idxe = zero_i
                    idxr = zero_i
                    for j in range(16):
                        recj = rbufs[s][pl.ds(j * 16, 16)]
                        ei = jnp.full((16,), jnp.int32(recj[5]), jnp.int32)
                        ri = jnp.full((16,), jnp.int32(recj[1]), jnp.int32)
                        idxe = jnp.where(iot == j, ei, idxe)
                        idxr = jnp.where(iot == j, ri, idxr)
                    idxe = jnp.clip(idxe, 0, e - 1)
                    idxr = jnp.clip(idxr, 0, n2 - 1)
                    pltpu.async_copy(w_hbm.at[idxe], wgs[s], sems[s])
                    pltpu.async_copy(x_hbm.at[idxr], xgs[s], sems[s])
                    pltpu.async_copy(v_hbm.at[idxr], vgs[s], sems[s])

                def wait_slot(s):
                    for _ in range(3):
                        pltpu.make_async_copy(
                            x_hbm.at[zero_i], xgs[s], sems[s]).wait()

                def compute_chunk(s, ch2):
                    nvalid = jnp.minimum(cnt - ch2 * 16, 16)
                    wg, xg, vg = wgs[s], xgs[s], vgs[s]

                    def compute(j, c0):
                        recj = rbufs[s][pl.ds(j * 16, 16)]
                        row = jnp.int32(recj[0]) - base
                        sc = recj[6]
                        scv = jnp.full((16,), sc, jnp.float32)
                        dvecs = [jnp.full((16,), recj[2 + kk] * sc,
                                          jnp.float32)
                                 for kk in range(3)]
                        for c in range(8):
                            co = c * 16
                            wv0 = wg[j, pl.ds(co, 16)]
                            wv1 = wg[j, pl.ds(H + co, 16)]
                            wv2 = wg[j, pl.ds(2 * H + co, 16)]
                            xv0 = xg[j, pl.ds(co, 16)]
                            xv1 = xg[j, pl.ds(H + co, 16)]
                            xv2 = xg[j, pl.ds(2 * H + co, 16)]
                            dv1 = wv1 * xv1
                            dv2 = wv2 * xv2 * scv
                            plsc.addupdate(
                                acc.at[row, pl.ds(co, 16)], wv0 * xv0 * scv)
                            for kk in range(3):
                                vj = vg[j, pl.ds(kk * H + co, 16)]
                                plsc.addupdate(
                                    acc.at[row, pl.ds(H + kk * H + co, 16)],
                                    dv1 * dvecs[kk] + dv2 * vj)
                        return c0

                    lax.fori_loop(0, nvalid, compute, 0)

                pl.when(nch > 0)(lambda: fire(0, 0))

                def pair_body(p, c):
                    ch0 = p * 2
                    wait_slot(0)
                    pl.when(ch0 + 1 < nch)(lambda: fire(1, ch0 + 1))
                    compute_chunk(0, ch0)

                    @pl.when(ch0 + 1 < nch)
                    def _():
                        wait_slot(1)
                        pl.when(ch0 + 2 < nch)(lambda: fire(0, ch0 + 2))
                        compute_chunk(1, ch0 + 1)

                    return c

                lax.fori_loop(0, (nch + 1) >> 1, pair_body, 0)
                pltpu.sync_copy(acc, out_hbm.at[pl.ds(base, NB)])

    return k(sorted2d, counts, x2d, v2d, w2d)


def _update(s2d, v2d, dsum, dvsum, W_vmix, W_mix1, b_mix1, W_mix2, b_mix2):
    n = s2d.shape[0]
    bs = 400

    def body(s_ref, v_ref, ds_ref, dv_ref, wv_ref, w1_ref, b1_ref, w2_ref,
             b2_ref, so_ref, vo_ref):
        s1 = s_ref[...] + _clip(ds_ref[...])
        v1 = v_ref[...] + _clip(dv_ref[...])
        wv = wv_ref[...]
        v1k = [v1[:, kk * H:(kk + 1) * H] for kk in range(3)]
        vm = [jnp.dot(vk, wv, preferred_element_type=jnp.float32) for vk in v1k]
        v_l = [m[:, :H] for m in vm]
        v_r = [m[:, H:] for m in vm]
        nsq = v_r[0] * v_r[0] + v_r[1] * v_r[1] + v_r[2] * v_r[2]
        v_norm = jnp.sqrt(nsq + EPS)
        w1 = w1_ref[...]
        h = (jnp.dot(s1, w1[:H, :], preferred_element_type=jnp.float32)
             + jnp.dot(v_norm, w1[H:, :], preferred_element_type=jnp.float32)
             + b1_ref[...])
        h = h * jax.nn.sigmoid(h)
        m = jnp.dot(h, w2_ref[...], preferred_element_type=jnp.float32)
        m = m + b2_ref[...]
        ds2 = m[:, :H]
        dvu_g = m[:, H:2 * H]
        dsv_g = m[:, 2 * H:]
        dot_rl = v_r[0] * v_l[0] + v_r[1] * v_l[1] + v_r[2] * v_l[2]
        so_ref[...] = s1 + _clip(ds2 + dsv_g * dot_rl)
        vo_ref[...] = jnp.concatenate(
            [v1k[kk] + _clip(v_l[kk] * dvu_g) for kk in range(3)], axis=1)

    return pl.pallas_call(
        body,
        grid=(n // bs,),
        in_specs=[
            pl.BlockSpec((bs, H), lambda i: (i, 0)),
            pl.BlockSpec((bs, H3), lambda i: (i, 0)),
            pl.BlockSpec((bs, H), lambda i: (i, 0)),
            pl.BlockSpec((bs, H3), lambda i: (i, 0)),
            pl.BlockSpec((H, 2 * H), lambda i: (0, 0)),
            pl.BlockSpec((2 * H, H), lambda i: (0, 0)),
            pl.BlockSpec((1, H), lambda i: (0, 0)),
            pl.BlockSpec((H, H3), lambda i: (0, 0)),
            pl.BlockSpec((1, H3), lambda i: (0, 0)),
        ],
        out_specs=[
            pl.BlockSpec((bs, H), lambda i: (i, 0)),
            pl.BlockSpec((bs, H3), lambda i: (i, 0)),
        ],
        out_shape=[
            jax.ShapeDtypeStruct((n, H), jnp.float32),
            jax.ShapeDtypeStruct((n, H3), jnp.float32),
        ],
    )(s2d, v2d, dsum, dvsum, W_vmix, W_mix1, b_mix1.reshape(1, H), W_mix2,
      b_mix2.reshape(1, H3))


def kernel(s, v, dir_ij, Wij, senders, receivers, W_int1, b_int1, W_int2,
           b_int2, W_vmix, W_mix1, b_mix1, W_mix2, b_mix2):
    n = s.shape[0]
    e = senders.shape[0]
    s2d = s.reshape(n, H)
    v2d = v.reshape(n, H3)
    w2d = Wij.reshape(e, H3)

    x2d = _mlp1(s2d, W_int1, b_int1, W_int2, b_int2)

    counts = _sc_hist(senders)
    sorted2d = _sc_scatter(senders, receivers, dir_ij[:, 0], dir_ij[:, 1],
                           dir_ij[:, 2], counts)
    agg = _sc_main(sorted2d, counts, x2d, v2d, w2d)
    dsum = agg[:n, :H]
    dvsum = agg[:n, H:]

    s_out, v_out = _update(s2d, v2d, dsum, dvsum, W_vmix, W_mix1, b_mix1,
                           W_mix2, b_mix2)
    return (s_out.reshape(n, 1, H), v_out.reshape(n, 3, H))
